# baseline (speedup 1.0000x reference)
import functools

import jax
import jax.numpy as jnp
from jax import lax
from jax.experimental import pallas as pl
from jax.experimental.pallas import tpu as pltpu
from jax.experimental.pallas import tpu_sc as plsc

B = 8192
N_ANCHORS = 2
N_VARS = 3
N = B * (N_ANCHORS + N_VARS)
E = 24576
D = 128
R = 64
NB = 10

NC = 2
NS = 16
NT = NC * NS
CH = 128

_MESH = plsc.VectorSubcoreMesh(core_axis_name="c", subcore_axis_name="s")


def _row_gather(table, idx_rows, n_out):
    nchunks = idx_rows.shape[1]
    per_tile = nchunks * CH

    @functools.partial(
        pl.kernel,
        out_type=jax.ShapeDtypeStruct((n_out, D), jnp.float32),
        mesh=_MESH,
        scratch_types=[
            pltpu.VMEM((nchunks, CH), jnp.int32),
            pltpu.VMEM((per_tile, D), jnp.float32),
        ],
    )
    def k(table_hbm, idx_hbm, out_hbm, idx_v, buf_v):
        wid = lax.axis_index("c") * NS + lax.axis_index("s")
        pltpu.sync_copy(idx_hbm.at[wid], idx_v)
        for j in range(nchunks):
            pltpu.sync_copy(table_hbm.at[idx_v.at[j]],
                            buf_v.at[pl.ds(j * CH, CH)])
        pltpu.sync_copy(buf_v, out_hbm.at[pl.ds(wid * per_tile, per_tile)])

    return k(table, idx_rows)


_CPARTS = 8
_PC = N * R // _CPARTS
_ZPT = (_PC + CH) // NS


def _pair_counts(cnt_idx, out_idx, zeros_flat, ones_v):
    @functools.partial(
        pl.kernel,
        out_type=jax.ShapeDtypeStruct((E + CH,), jnp.float32),
        mesh=_MESH,
        scratch_types=[
            pltpu.VMEM_SHARED((_PC + CH,), jnp.float32),
            pltpu.VMEM((E // NS // CH, CH), jnp.int32),
            pltpu.VMEM((E // NS // CH, CH), jnp.int32),
            pltpu.VMEM((CH,), jnp.float32),
            pltpu.VMEM((CH,), jnp.float32),
            pltpu.VMEM((_ZPT,), jnp.float32),
        ],
    )
    def k(cidx_hbm, oidx_hbm, zflat_hbm, ones_hbm, cnt_hbm, shared, idx_v,
          oidx_v, ones_vv, val_v, zbuf_v):
        c = lax.axis_index("c")
        t = lax.axis_index("s")
        nchunks = E // NS // CH
        pltpu.sync_copy(ones_hbm, ones_vv)
        pltpu.sync_copy(zflat_hbm, zbuf_v)
        for p in range(_CPARTS // 2):
            g = c * (_CPARTS // 2) + p
            pltpu.sync_copy(zbuf_v, shared.at[pl.ds(t * _ZPT, _ZPT)])
            plsc.subcore_barrier()
            pltpu.sync_copy(cidx_hbm.at[g * NS + t], idx_v)
            pltpu.sync_copy(oidx_hbm.at[g * NS + t], oidx_v)
            for j in range(nchunks):
                pltpu.sync_copy(ones_vv, shared.at[idx_v.at[j]], add=True)
            plsc.subcore_barrier()
            for j in range(nchunks):
                pltpu.sync_copy(shared.at[idx_v.at[j]], val_v)
                pltpu.sync_copy(val_v, cnt_hbm.at[oidx_v.at[j]])
            plsc.subcore_barrier()

    return k(cnt_idx, out_idx, zeros_flat, ones_v)


_APARTS = 8
_PART = N // _APARTS
_PROWS = _PART + CH
_ZROWS = _PROWS // NS
_RPT = _PART // NS


def _agg_h(msg, xroot, agg_idx, zeros_rows):
    @functools.partial(
        pl.kernel,
        out_type=jax.ShapeDtypeStruct((N // 64, 64, D), jnp.float32),
        mesh=_MESH,
        scratch_types=[
            pltpu.VMEM_SHARED((_PROWS, D), jnp.float32),
            pltpu.VMEM((E // NS // CH, CH), jnp.int32),
            pltpu.VMEM((CH, D), jnp.float32),
            pltpu.VMEM((CH, D), jnp.float32),
            pltpu.VMEM((CH, D), jnp.float32),
            pltpu.VMEM((CH, D), jnp.float32),
        ],
    )
    def k(msg_hbm, xroot_hbm, aidx_hbm, zrows_hbm, h_hbm,
          acc, idx_v, data_v, zbuf_v, rbuf_v, xrbuf_v):
        c = lax.axis_index("c")
        t = lax.axis_index("s")
        nchunks = E // NS // CH
        pltpu.sync_copy(zrows_hbm, zbuf_v)
        for p in range(_APARTS // 2):
            g = c * (_APARTS // 2) + p
            for j in range(_ZROWS // CH):
                pltpu.sync_copy(zbuf_v,
                                acc.at[pl.ds(t * _ZROWS + j * CH, CH)])
            if _ZROWS % CH:
                pltpu.sync_copy(
                    zbuf_v.at[pl.ds(0, _ZROWS % CH)],
                    acc.at[pl.ds(t * _ZROWS + (_ZROWS // CH) * CH,
                                 _ZROWS % CH)])
            plsc.subcore_barrier()
            pltpu.sync_copy(aidx_hbm.at[g * NS + t], idx_v)
            for j in range(nchunks):
                pltpu.sync_copy(msg_hbm.at[pl.ds(t * (E // NS) + j * CH, CH)],
                                data_v)
                pltpu.sync_copy(data_v, acc.at[idx_v.at[j]], add=True)
            plsc.subcore_barrier()
            def row_body(r, carry):
                for l in range(D // 16):
                    sl = pl.ds(l * 16, 16)
                    rbuf_v[r, sl] = jnp.maximum(
                        rbuf_v[r, sl] + xrbuf_v[r, sl], 0.0)
                return carry

            for j in range(_RPT // 64):
                pltpu.sync_copy(acc.at[pl.ds(t * _RPT + j * 64, 64)],
                                rbuf_v.at[pl.ds(0, 64)])
                pltpu.sync_copy(xroot_hbm.at[(g * NS + t) * (_RPT // 64) + j],
                                xrbuf_v.at[pl.ds(0, 64)])
                lax.fori_loop(0, 64, row_body, 0)
                pltpu.sync_copy(rbuf_v.at[pl.ds(0, 64)],
                                h_hbm.at[(g * NS + t) * (_RPT // 64) + j])
            plsc.subcore_barrier()

    return k(msg, xroot, agg_idx, zeros_rows)


_BHALF = B // 2
_BROWS = _BHALF + CH
_BZ = _BROWS // NS


def _pool_scatter(h, pool_idx, zeros_rows):
    @functools.partial(
        pl.kernel,
        out_type=jax.ShapeDtypeStruct((2 * B, D), jnp.float32),
        mesh=_MESH,
        scratch_types=[
            pltpu.VMEM_SHARED((_BROWS, D), jnp.float32),
            pltpu.VMEM((N // 2 // NS // CH, CH), jnp.int32),
            pltpu.VMEM((CH, D), jnp.float32),
            pltpu.VMEM((CH, D), jnp.float32),
        ],
    )
    def k(h_hbm, pidx_hbm, zrows_hbm, out_hbm, pool, idx_v, data_v, zbuf_v):
        c = lax.axis_index("c")
        t = lax.axis_index("s")
        nchunks = N // 2 // NS // CH
        pltpu.sync_copy(zrows_hbm, zbuf_v)
        for q in range(2):
            for j in range(_BZ // CH):
                pltpu.sync_copy(zbuf_v,
                                pool.at[pl.ds(t * _BZ + j * CH, CH)])
            if _BZ % CH:
                pltpu.sync_copy(
                    zbuf_v.at[pl.ds(0, _BZ % CH)],
                    pool.at[pl.ds(t * _BZ + (_BZ // CH) * CH, _BZ % CH)])
            plsc.subcore_barrier()
            pltpu.sync_copy(pidx_hbm.at[q * NT + c * NS + t], idx_v)
            for j in range(nchunks):
                pltpu.sync_copy(
                    h_hbm.at[
                        pl.ds(c * (N // 2) + t * nchunks * CH + j * CH, CH)],
                    data_v)
                pltpu.sync_copy(data_v, pool.at[idx_v.at[j]], add=True)
            plsc.subcore_barrier()
            rpt = _BHALF // NS
            for j in range(rpt // CH):
                pltpu.sync_copy(pool.at[pl.ds(t * rpt + j * CH, CH)], data_v)
                pltpu.sync_copy(
                    data_v,
                    out_hbm.at[
                        pl.ds(c * B + q * _BHALF + t * rpt + j * CH, CH)])
            plsc.subcore_barrier()

    return k(h, pool_idx, zeros_rows)


_BE = 1024


def _msg_body(xs_ref, et_ref, cnt_ref, comp_ref, basesf_ref, out_ref):
    et = et_ref[0]
    iot = lax.broadcasted_iota(jnp.int32, (1, R), 1)
    onehot = (et == iot).astype(jnp.float32)
    cs = jnp.dot(onehot, comp_ref[...],
                 preferred_element_type=jnp.float32)
    xsb = jnp.dot(xs_ref[...], basesf_ref[...],
                  preferred_element_type=jnp.float32)
    acc = xsb[:, 0:D] * cs[:, 0:1]
    for b in range(1, NB):
        acc = acc + xsb[:, b * D:(b + 1) * D] * cs[:, b:b + 1]
    norm = 1.0 / jnp.maximum(cnt_ref[0], 1.0)
    out_ref[...] = acc * norm


def _tc_msg(xs, et3, cnt3, comp, basesf):
    return pl.pallas_call(
        _msg_body,
        grid=(E // _BE,),
        in_specs=[
            pl.BlockSpec((_BE, D), lambda i: (i, 0)),
            pl.BlockSpec((1, _BE, 1), lambda i: (i, 0, 0)),
            pl.BlockSpec((1, _BE, 1), lambda i: (i, 0, 0)),
            pl.BlockSpec((R, NB), lambda i: (0, 0)),
            pl.BlockSpec((D, NB * D), lambda i: (0, 0)),
        ],
        out_specs=pl.BlockSpec((_BE, D), lambda i: (i, 0)),
        out_shape=jax.ShapeDtypeStruct((E, D), jnp.float32),
    )(xs, et3, cnt3, comp, basesf)


_BN = 2048


def _xroot_body(x_ref, root_ref, bias_ref, out_ref):
    out_ref[...] = jnp.dot(x_ref[...], root_ref[...],
                           preferred_element_type=jnp.float32) + bias_ref[...]


def _tc_xroot(x, root, bias2):
    return pl.pallas_call(
        _xroot_body,
        grid=(N // _BN,),
        in_specs=[
            pl.BlockSpec((_BN, D), lambda i: (i, 0)),
            pl.BlockSpec((D, D), lambda i: (0, 0)),
            pl.BlockSpec((1, D), lambda i: (0, 0)),
        ],
        out_specs=pl.BlockSpec((_BN, D), lambda i: (i, 0)),
        out_shape=jax.ShapeDtypeStruct((N, D), jnp.float32),
    )(x, root, bias2)


def _cos_body(p2_ref, tgt_ref, out_ref):
    pooled = p2_ref[0:B, :] + p2_ref[B:2 * B, :]
    tgt = tgt_ref[...]
    num = jnp.sum(pooled * tgt, axis=1, keepdims=True)
    den = jnp.sqrt(jnp.sum(pooled * pooled, axis=1, keepdims=True)) * \
        jnp.sqrt(jnp.sum(tgt * tgt, axis=1, keepdims=True))
    out_ref[...] = num / jnp.maximum(den, 1e-8)


def _tc_cos(pooled2, tgt):
    return pl.pallas_call(
        _cos_body,
        out_shape=jax.ShapeDtypeStruct((B, 1), jnp.float32),
    )(pooled2, tgt)


def kernel(anchor_ids, var_ids, edge_index, edge_type, batch_vec,
           target_nodes, node_emb, mode_emb, comp, bases, root, bias):
    anchor_ids = anchor_ids.astype(jnp.int32)
    target_nodes = target_nodes.astype(jnp.int32)
    src = edge_index[0].astype(jnp.int32)
    dst = edge_index[1].astype(jnp.int32)
    et = edge_type.astype(jnp.int32)
    bv = batch_vec.astype(jnp.int32)

    zeros_rows = jnp.zeros((CH, D), jnp.float32)
    zeros_flat = jnp.zeros((_ZPT,), jnp.float32)
    ones_v = jnp.ones((CH,), jnp.float32)

    gidx = jnp.concatenate([anchor_ids.reshape(-1), target_nodes])
    rows = _row_gather(node_emb, gidx.reshape(NT, 6, CH), B * 3)
    anch = rows[:B * N_ANCHORS].reshape(B, N_ANCHORS, D)
    tgt = rows[B * N_ANCHORS:]
    onehot3 = (var_ids[:, None] ==
               jnp.arange(mode_emb.shape[0], dtype=var_ids.dtype)[None, :])
    var3 = jnp.dot(onehot3.astype(jnp.float32), mode_emb)
    x = jnp.concatenate(
        [anch, jnp.broadcast_to(var3[None], (B, N_VARS, D))],
        axis=1).reshape(N, D)

    xs = _row_gather(x, src.reshape(NT, 6, CH), E)

    comb = dst * R + et
    eid = jnp.arange(E, dtype=jnp.int32)
    cparts, oparts = [], []
    for g in range(_CPARTS):
        in_part = (comb >= g * _PC) & (comb < (g + 1) * _PC)
        cparts.append(jnp.where(in_part, comb - g * _PC, _PC))
        oparts.append(jnp.where(in_part, eid, E))
    cnt_idx = jnp.stack(cparts).reshape(_CPARTS * NS, 12, CH)
    out_idx = jnp.stack(oparts).reshape(_CPARTS * NS, 12, CH)
    cnt_e = _pair_counts(cnt_idx, out_idx, zeros_flat, ones_v)[:E]

    basesf = jnp.transpose(bases, (1, 0, 2)).reshape(D, NB * D)
    msg = _tc_msg(xs, et.reshape(E // _BE, _BE, 1),
                  cnt_e.reshape(E // _BE, _BE, 1), comp, basesf)

    xroot = _tc_xroot(x, root, bias.reshape(1, D))

    aparts = []
    for g in range(_APARTS):
        loc = dst - g * _PART
        aparts.append(jnp.where((dst >= g * _PART) & (dst < (g + 1) * _PART),
                                loc, _PART))
    agg_idx = jnp.stack(aparts).reshape(_APARTS * NS, 12, CH)
    h = _agg_h(msg, xroot.reshape(N // 64, 64, D), agg_idx,
               zeros_rows).reshape(N, D)

    pparts = []
    for q in range(2):
        loc = bv - q * _BHALF
        pparts.append(jnp.where((bv >= q * _BHALF) & (bv < (q + 1) * _BHALF),
                                loc, _BHALF))
    pool_idx = jnp.stack(pparts).reshape(2 * NT, 10, CH)
    pooled2 = _pool_scatter(h, pool_idx, zeros_rows)

    out2 = _tc_cos(pooled2, tgt)
    return out2.reshape(B)


# speedup vs baseline: 1.0023x; 1.0023x over previous
import functools

import jax
import jax.numpy as jnp
from jax import lax
from jax.experimental import pallas as pl
from jax.experimental.pallas import tpu as pltpu
from jax.experimental.pallas import tpu_sc as plsc

B = 8192
N_ANCHORS = 2
N_VARS = 3
N = B * (N_ANCHORS + N_VARS)
E = 24576
D = 128
R = 64
NB = 10

NC = 2
NS = 16
NT = NC * NS
CH = 128

_MESH = plsc.VectorSubcoreMesh(core_axis_name="c", subcore_axis_name="s")


def _row_gather(table, idx_rows, n_out):
    nchunks = idx_rows.shape[1]
    per_tile = nchunks * CH

    @functools.partial(
        pl.kernel,
        out_type=jax.ShapeDtypeStruct((n_out, D), jnp.float32),
        mesh=_MESH,
        scratch_types=[
            pltpu.VMEM((nchunks, CH), jnp.int32),
            pltpu.VMEM((per_tile, D), jnp.float32),
        ],
    )
    def k(table_hbm, idx_hbm, out_hbm, idx_v, buf_v):
        wid = lax.axis_index("c") * NS + lax.axis_index("s")
        pltpu.sync_copy(idx_hbm.at[wid], idx_v)
        for j in range(nchunks):
            pltpu.sync_copy(table_hbm.at[idx_v.at[j]],
                            buf_v.at[pl.ds(j * CH, CH)])
        pltpu.sync_copy(buf_v, out_hbm.at[pl.ds(wid * per_tile, per_tile)])

    return k(table, idx_rows)


_CPARTS = 8
_PC = N * R // _CPARTS
_ZPT = (_PC + CH) // NS


def _pair_counts(cnt_idx, out_idx, zeros_flat, ones_v):
    @functools.partial(
        pl.kernel,
        out_type=jax.ShapeDtypeStruct((E + CH,), jnp.float32),
        mesh=_MESH,
        scratch_types=[
            pltpu.VMEM_SHARED((_PC + CH,), jnp.float32),
            pltpu.VMEM((E // NS // CH, CH), jnp.int32),
            pltpu.VMEM((E // NS // CH, CH), jnp.int32),
            pltpu.VMEM((CH,), jnp.float32),
            pltpu.VMEM((CH,), jnp.float32),
            pltpu.VMEM((_ZPT,), jnp.float32),
        ],
    )
    def k(cidx_hbm, oidx_hbm, zflat_hbm, ones_hbm, cnt_hbm, shared, idx_v,
          oidx_v, ones_vv, val_v, zbuf_v):
        c = lax.axis_index("c")
        t = lax.axis_index("s")
        nchunks = E // NS // CH
        pltpu.sync_copy(ones_hbm, ones_vv)
        pltpu.sync_copy(zflat_hbm, zbuf_v)
        for p in range(_CPARTS // 2):
            g = c * (_CPARTS // 2) + p
            pltpu.sync_copy(zbuf_v, shared.at[pl.ds(t * _ZPT, _ZPT)])
            plsc.subcore_barrier()
            pltpu.sync_copy(cidx_hbm.at[g * NS + t], idx_v)
            pltpu.sync_copy(oidx_hbm.at[g * NS + t], oidx_v)
            for j in range(nchunks):
                pltpu.sync_copy(ones_vv, shared.at[idx_v.at[j]], add=True)
            plsc.subcore_barrier()
            for j in range(nchunks):
                pltpu.sync_copy(shared.at[idx_v.at[j]], val_v)
                pltpu.sync_copy(val_v, cnt_hbm.at[oidx_v.at[j]])
            plsc.subcore_barrier()

    return k(cnt_idx, out_idx, zeros_flat, ones_v)


_APARTS = 8
_PART = N // _APARTS
_PROWS = _PART + CH
_ZROWS = _PROWS // NS
_RPT = _PART // NS


def _agg_scatter(msg, agg_idx, zeros_rows):
    @functools.partial(
        pl.kernel,
        out_type=jax.ShapeDtypeStruct((N // 64, 64, D), jnp.float32),
        mesh=_MESH,
        scratch_types=[
            pltpu.VMEM_SHARED((_PROWS, D), jnp.float32),
            pltpu.VMEM((E // NS // CH, CH), jnp.int32),
            pltpu.VMEM((CH, D), jnp.float32),
            pltpu.VMEM((CH, D), jnp.float32),
            pltpu.VMEM((CH, D), jnp.float32),
        ],
    )
    def k(msg_hbm, aidx_hbm, zrows_hbm, h_hbm,
          acc, idx_v, data_v, zbuf_v, rbuf_v):
        c = lax.axis_index("c")
        t = lax.axis_index("s")
        nchunks = E // NS // CH
        pltpu.sync_copy(zrows_hbm, zbuf_v)
        for p in range(_APARTS // 2):
            g = c * (_APARTS // 2) + p
            for j in range(_ZROWS // CH):
                pltpu.sync_copy(zbuf_v,
                                acc.at[pl.ds(t * _ZROWS + j * CH, CH)])
            if _ZROWS % CH:
                pltpu.sync_copy(
                    zbuf_v.at[pl.ds(0, _ZROWS % CH)],
                    acc.at[pl.ds(t * _ZROWS + (_ZROWS // CH) * CH,
                                 _ZROWS % CH)])
            plsc.subcore_barrier()
            pltpu.sync_copy(aidx_hbm.at[g * NS + t], idx_v)
            for j in range(nchunks):
                pltpu.sync_copy(msg_hbm.at[pl.ds(t * (E // NS) + j * CH, CH)],
                                data_v)
                pltpu.sync_copy(data_v, acc.at[idx_v.at[j]], add=True)
            plsc.subcore_barrier()
            for j in range(_RPT // 64):
                pltpu.sync_copy(acc.at[pl.ds(t * _RPT + j * 64, 64)],
                                rbuf_v.at[pl.ds(0, 64)])
                pltpu.sync_copy(rbuf_v.at[pl.ds(0, 64)],
                                h_hbm.at[(g * NS + t) * (_RPT // 64) + j])
            plsc.subcore_barrier()

    return k(msg, agg_idx, zeros_rows)


_BHALF = B // 2
_BROWS = _BHALF + CH
_BZ = _BROWS // NS


def _pool_scatter(h, pool_idx, zeros_rows):
    @functools.partial(
        pl.kernel,
        out_type=jax.ShapeDtypeStruct((2 * B, D), jnp.float32),
        mesh=_MESH,
        scratch_types=[
            pltpu.VMEM_SHARED((_BROWS, D), jnp.float32),
            pltpu.VMEM((N // 2 // NS // CH, CH), jnp.int32),
            pltpu.VMEM((CH, D), jnp.float32),
            pltpu.VMEM((CH, D), jnp.float32),
        ],
    )
    def k(h_hbm, pidx_hbm, zrows_hbm, out_hbm, pool, idx_v, data_v, zbuf_v):
        c = lax.axis_index("c")
        t = lax.axis_index("s")
        nchunks = N // 2 // NS // CH
        pltpu.sync_copy(zrows_hbm, zbuf_v)
        for q in range(2):
            for j in range(_BZ // CH):
                pltpu.sync_copy(zbuf_v,
                                pool.at[pl.ds(t * _BZ + j * CH, CH)])
            if _BZ % CH:
                pltpu.sync_copy(
                    zbuf_v.at[pl.ds(0, _BZ % CH)],
                    pool.at[pl.ds(t * _BZ + (_BZ // CH) * CH, _BZ % CH)])
            plsc.subcore_barrier()
            pltpu.sync_copy(pidx_hbm.at[q * NT + c * NS + t], idx_v)
            for j in range(nchunks):
                pltpu.sync_copy(
                    h_hbm.at[
                        pl.ds(c * (N // 2) + t * nchunks * CH + j * CH, CH)],
                    data_v)
                pltpu.sync_copy(data_v, pool.at[idx_v.at[j]], add=True)
            plsc.subcore_barrier()
            rpt = _BHALF // NS
            for j in range(rpt // CH):
                pltpu.sync_copy(pool.at[pl.ds(t * rpt + j * CH, CH)], data_v)
                pltpu.sync_copy(
                    data_v,
                    out_hbm.at[
                        pl.ds(c * B + q * _BHALF + t * rpt + j * CH, CH)])
            plsc.subcore_barrier()

    return k(h, pool_idx, zeros_rows)


_BE = 1024


def _msg_body(xs_ref, et_ref, cnt_ref, comp_ref, basesf_ref, out_ref):
    et = et_ref[0]
    iot = lax.broadcasted_iota(jnp.int32, (1, R), 1)
    onehot = (et == iot).astype(jnp.float32)
    cs = jnp.dot(onehot, comp_ref[...],
                 preferred_element_type=jnp.float32)
    xsb = jnp.dot(xs_ref[...], basesf_ref[...],
                  preferred_element_type=jnp.float32)
    acc = xsb[:, 0:D] * cs[:, 0:1]
    for b in range(1, NB):
        acc = acc + xsb[:, b * D:(b + 1) * D] * cs[:, b:b + 1]
    norm = 1.0 / jnp.maximum(cnt_ref[0], 1.0)
    out_ref[...] = acc * norm


def _tc_msg(xs, et3, cnt3, comp, basesf):
    return pl.pallas_call(
        _msg_body,
        grid=(E // _BE,),
        in_specs=[
            pl.BlockSpec((_BE, D), lambda i: (i, 0)),
            pl.BlockSpec((1, _BE, 1), lambda i: (i, 0, 0)),
            pl.BlockSpec((1, _BE, 1), lambda i: (i, 0, 0)),
            pl.BlockSpec((R, NB), lambda i: (0, 0)),
            pl.BlockSpec((D, NB * D), lambda i: (0, 0)),
        ],
        out_specs=pl.BlockSpec((_BE, D), lambda i: (i, 0)),
        out_shape=jax.ShapeDtypeStruct((E, D), jnp.float32),
    )(xs, et3, cnt3, comp, basesf)


_BN = 2048


def _h_body(agg_ref, x_ref, root_ref, bias_ref, out_ref):
    xr = jnp.dot(x_ref[...], root_ref[...],
                 preferred_element_type=jnp.float32)
    out_ref[...] = jnp.maximum(agg_ref[...] + xr + bias_ref[...], 0.0)


def _tc_h(agg, x, root, bias2):
    return pl.pallas_call(
        _h_body,
        grid=(N // _BN,),
        in_specs=[
            pl.BlockSpec((_BN, D), lambda i: (i, 0)),
            pl.BlockSpec((_BN, D), lambda i: (i, 0)),
            pl.BlockSpec((D, D), lambda i: (0, 0)),
            pl.BlockSpec((1, D), lambda i: (0, 0)),
        ],
        out_specs=pl.BlockSpec((_BN, D), lambda i: (i, 0)),
        out_shape=jax.ShapeDtypeStruct((N, D), jnp.float32),
    )(agg, x, root, bias2)


def _cos_body(p2_ref, tgt_ref, out_ref):
    pooled = p2_ref[0:B, :] + p2_ref[B:2 * B, :]
    tgt = tgt_ref[...]
    num = jnp.sum(pooled * tgt, axis=1, keepdims=True)
    den = jnp.sqrt(jnp.sum(pooled * pooled, axis=1, keepdims=True)) * \
        jnp.sqrt(jnp.sum(tgt * tgt, axis=1, keepdims=True))
    out_ref[...] = num / jnp.maximum(den, 1e-8)


def _tc_cos(pooled2, tgt):
    return pl.pallas_call(
        _cos_body,
        out_shape=jax.ShapeDtypeStruct((B, 1), jnp.float32),
    )(pooled2, tgt)


def kernel(anchor_ids, var_ids, edge_index, edge_type, batch_vec,
           target_nodes, node_emb, mode_emb, comp, bases, root, bias):
    anchor_ids = anchor_ids.astype(jnp.int32)
    target_nodes = target_nodes.astype(jnp.int32)
    src = edge_index[0].astype(jnp.int32)
    dst = edge_index[1].astype(jnp.int32)
    et = edge_type.astype(jnp.int32)
    bv = batch_vec.astype(jnp.int32)

    zeros_rows = jnp.zeros((CH, D), jnp.float32)
    zeros_flat = jnp.zeros((_ZPT,), jnp.float32)
    ones_v = jnp.ones((CH,), jnp.float32)

    gidx = jnp.concatenate([anchor_ids.reshape(-1), target_nodes])
    rows = _row_gather(node_emb, gidx.reshape(NT, 6, CH), B * 3)
    anch = rows[:B * N_ANCHORS].reshape(B, N_ANCHORS, D)
    tgt = rows[B * N_ANCHORS:]
    onehot3 = (var_ids[:, None] ==
               jnp.arange(mode_emb.shape[0], dtype=var_ids.dtype)[None, :])
    var3 = jnp.dot(onehot3.astype(jnp.float32), mode_emb)
    x = jnp.concatenate(
        [anch, jnp.broadcast_to(var3[None], (B, N_VARS, D))],
        axis=1).reshape(N, D)

    xs = _row_gather(x, src.reshape(NT, 6, CH), E)

    comb = dst * R + et
    eid = jnp.arange(E, dtype=jnp.int32)
    cparts, oparts = [], []
    for g in range(_CPARTS):
        in_part = (comb >= g * _PC) & (comb < (g + 1) * _PC)
        cparts.append(jnp.where(in_part, comb - g * _PC, _PC))
        oparts.append(jnp.where(in_part, eid, E))
    cnt_idx = jnp.stack(cparts).reshape(_CPARTS * NS, 12, CH)
    out_idx = jnp.stack(oparts).reshape(_CPARTS * NS, 12, CH)
    cnt_e = _pair_counts(cnt_idx, out_idx, zeros_flat, ones_v)[:E]

    basesf = jnp.transpose(bases, (1, 0, 2)).reshape(D, NB * D)
    msg = _tc_msg(xs, et.reshape(E // _BE, _BE, 1),
                  cnt_e.reshape(E // _BE, _BE, 1), comp, basesf)

    aparts = []
    for g in range(_APARTS):
        loc = dst - g * _PART
        aparts.append(jnp.where((dst >= g * _PART) & (dst < (g + 1) * _PART),
                                loc, _PART))
    agg_idx = jnp.stack(aparts).reshape(_APARTS * NS, 12, CH)
    agg = _agg_scatter(msg, agg_idx, zeros_rows).reshape(N, D)

    h = _tc_h(agg, x, root, bias.reshape(1, D))

    pparts = []
    for q in range(2):
        loc = bv - q * _BHALF
        pparts.append(jnp.where((bv >= q * _BHALF) & (bv < (q + 1) * _BHALF),
                                loc, _BHALF))
    pool_idx = jnp.stack(pparts).reshape(2 * NT, 10, CH)
    pooled2 = _pool_scatter(h, pool_idx, zeros_rows)

    out2 = _tc_cos(pooled2, tgt)
    return out2.reshape(B)


# speedup vs baseline: 54.7215x; 54.5958x over previous
import functools

import jax
import jax.numpy as jnp
from jax import lax
from jax.experimental import pallas as pl
from jax.experimental.pallas import tpu as pltpu
from jax.experimental.pallas import tpu_sc as plsc

B = 8192
N_ANCHORS = 2
N_VARS = 3
N = B * (N_ANCHORS + N_VARS)
E = 24576
D = 128
R = 64
NB = 10

NC = 2
NS = 16
NT = NC * NS
CH = 128

_MESH = plsc.VectorSubcoreMesh(core_axis_name="c", subcore_axis_name="s")


def _row_gather(table, idx_rows, n_out):
    nchunks = idx_rows.shape[1]
    per_tile = nchunks * CH

    @functools.partial(
        pl.kernel,
        out_type=jax.ShapeDtypeStruct((n_out, D), jnp.float32),
        mesh=_MESH,
        scratch_types=[
            pltpu.VMEM((nchunks, CH), jnp.int32),
            pltpu.VMEM((per_tile, D), jnp.float32),
        ],
    )
    def k(table_hbm, idx_hbm, out_hbm, idx_v, buf_v):
        wid = lax.axis_index("c") * NS + lax.axis_index("s")
        pltpu.sync_copy(idx_hbm.at[wid], idx_v)
        for j in range(nchunks):
            pltpu.sync_copy(table_hbm.at[idx_v.at[j]],
                            buf_v.at[pl.ds(j * CH, CH)])
        pltpu.sync_copy(buf_v, out_hbm.at[pl.ds(wid * per_tile, per_tile)])

    return k(table, idx_rows)


_CPARTS = 4
_PC = N * R // _CPARTS
_ZPT = (_PC + CH) // NS


def _pair_counts(cnt_idx, zeros_flat, ones_v):
    @functools.partial(
        pl.kernel,
        out_type=jax.ShapeDtypeStruct((N * R,), jnp.float32),
        mesh=_MESH,
        scratch_types=[
            pltpu.VMEM_SHARED((_PC + CH,), jnp.float32),
            pltpu.VMEM((E // NS // CH, CH), jnp.int32),
            pltpu.VMEM((CH,), jnp.float32),
            pltpu.VMEM((_ZPT,), jnp.float32),
        ],
    )
    def k(cidx_hbm, zflat_hbm, ones_hbm, cnt_hbm, shared, idx_v, ones_vv,
          buf_v):
        c = lax.axis_index("c")
        t = lax.axis_index("s")
        nchunks = E // NS // CH
        pltpu.sync_copy(ones_hbm, ones_vv)
        pltpu.sync_copy(zflat_hbm, buf_v)
        for p in range(_CPARTS // 2):
            g = c * (_CPARTS // 2) + p
            pltpu.sync_copy(buf_v, shared.at[pl.ds(t * _ZPT, _ZPT)])
            plsc.subcore_barrier()
            pltpu.sync_copy(cidx_hbm.at[g * NS + t], idx_v)
            for j in range(nchunks):
                pltpu.sync_copy(ones_vv, shared.at[idx_v.at[j]], add=True)
            plsc.subcore_barrier()
            per_tile = _PC // NS
            pltpu.sync_copy(shared.at[pl.ds(t * per_tile, per_tile)],
                            buf_v.at[pl.ds(0, per_tile)])
            pltpu.sync_copy(
                buf_v.at[pl.ds(0, per_tile)],
                cnt_hbm.at[pl.ds(g * _PC + t * per_tile, per_tile)])
            plsc.subcore_barrier()
            pltpu.sync_copy(zflat_hbm, buf_v)

    return k(cnt_idx, zeros_flat, ones_v)


def _elem_gather(table, idx_rows):
    nchunks = idx_rows.shape[1]
    per_tile = nchunks * CH

    @functools.partial(
        pl.kernel,
        out_type=jax.ShapeDtypeStruct((E,), jnp.float32),
        mesh=_MESH,
        scratch_types=[
            pltpu.VMEM((nchunks, CH), jnp.int32),
            pltpu.VMEM((per_tile,), jnp.float32),
        ],
    )
    def k(table_hbm, idx_hbm, out_hbm, idx_v, buf_v):
        wid = lax.axis_index("c") * NS + lax.axis_index("s")
        pltpu.sync_copy(idx_hbm.at[wid], idx_v)
        for j in range(nchunks):
            pltpu.sync_copy(table_hbm.at[idx_v.at[j]],
                            buf_v.at[pl.ds(j * CH, CH)])
        pltpu.sync_copy(buf_v, out_hbm.at[pl.ds(wid * per_tile, per_tile)])

    return k(table, idx_rows)


_APARTS = 8
_PART = N // _APARTS
_PROWS = _PART + CH
_ZROWS = _PROWS // NS
_RPT = _PART // NS


def _agg_scatter(msg, agg_idx, zeros_rows):
    @functools.partial(
        pl.kernel,
        out_type=jax.ShapeDtypeStruct((N // 64, 64, D), jnp.float32),
        mesh=_MESH,
        scratch_types=[
            pltpu.VMEM_SHARED((_PROWS, D), jnp.float32),
            pltpu.VMEM((E // NS // CH, CH), jnp.int32),
            pltpu.VMEM((CH, D), jnp.float32),
            pltpu.VMEM((CH, D), jnp.float32),
            pltpu.VMEM((CH, D), jnp.float32),
        ],
    )
    def k(msg_hbm, aidx_hbm, zrows_hbm, h_hbm,
          acc, idx_v, data_v, zbuf_v, rbuf_v):
        c = lax.axis_index("c")
        t = lax.axis_index("s")
        nchunks = E // NS // CH
        pltpu.sync_copy(zrows_hbm, zbuf_v)
        for p in range(_APARTS // 2):
            g = c * (_APARTS // 2) + p
            for j in range(_ZROWS // CH):
                pltpu.sync_copy(zbuf_v,
                                acc.at[pl.ds(t * _ZROWS + j * CH, CH)])
            if _ZROWS % CH:
                pltpu.sync_copy(
                    zbuf_v.at[pl.ds(0, _ZROWS % CH)],
                    acc.at[pl.ds(t * _ZROWS + (_ZROWS // CH) * CH,
                                 _ZROWS % CH)])
            plsc.subcore_barrier()
            pltpu.sync_copy(aidx_hbm.at[g * NS + t], idx_v)
            for j in range(nchunks):
                pltpu.sync_copy(msg_hbm.at[pl.ds(t * (E // NS) + j * CH, CH)],
                                data_v)
                pltpu.sync_copy(data_v, acc.at[idx_v.at[j]], add=True)
            plsc.subcore_barrier()
            for j in range(_RPT // 64):
                pltpu.sync_copy(acc.at[pl.ds(t * _RPT + j * 64, 64)],
                                rbuf_v.at[pl.ds(0, 64)])
                pltpu.sync_copy(rbuf_v.at[pl.ds(0, 64)],
                                h_hbm.at[(g * NS + t) * (_RPT // 64) + j])
            plsc.subcore_barrier()

    return k(msg, agg_idx, zeros_rows)


_BHALF = B // 2
_BROWS = _BHALF + CH
_BZ = _BROWS // NS


def _pool_scatter(h, pool_idx, zeros_rows):
    @functools.partial(
        pl.kernel,
        out_type=jax.ShapeDtypeStruct((2 * B, D), jnp.float32),
        mesh=_MESH,
        scratch_types=[
            pltpu.VMEM_SHARED((_BROWS, D), jnp.float32),
            pltpu.VMEM((N // 2 // NS // CH, CH), jnp.int32),
            pltpu.VMEM((CH, D), jnp.float32),
            pltpu.VMEM((CH, D), jnp.float32),
        ],
    )
    def k(h_hbm, pidx_hbm, zrows_hbm, out_hbm, pool, idx_v, data_v, zbuf_v):
        c = lax.axis_index("c")
        t = lax.axis_index("s")
        nchunks = N // 2 // NS // CH
        pltpu.sync_copy(zrows_hbm, zbuf_v)
        for q in range(2):
            for j in range(_BZ // CH):
                pltpu.sync_copy(zbuf_v,
                                pool.at[pl.ds(t * _BZ + j * CH, CH)])
            if _BZ % CH:
                pltpu.sync_copy(
                    zbuf_v.at[pl.ds(0, _BZ % CH)],
                    pool.at[pl.ds(t * _BZ + (_BZ // CH) * CH, _BZ % CH)])
            plsc.subcore_barrier()
            pltpu.sync_copy(pidx_hbm.at[q * NT + c * NS + t], idx_v)
            for j in range(nchunks):
                pltpu.sync_copy(
                    h_hbm.at[
                        pl.ds(c * (N // 2) + t * nchunks * CH + j * CH, CH)],
                    data_v)
                pltpu.sync_copy(data_v, pool.at[idx_v.at[j]], add=True)
            plsc.subcore_barrier()
            rpt = _BHALF // NS
            for j in range(rpt // CH):
                pltpu.sync_copy(pool.at[pl.ds(t * rpt + j * CH, CH)], data_v)
                pltpu.sync_copy(
                    data_v,
                    out_hbm.at[
                        pl.ds(c * B + q * _BHALF + t * rpt + j * CH, CH)])
            plsc.subcore_barrier()

    return k(h, pool_idx, zeros_rows)


_BE = 1024


def _msg_body(xs_ref, et_ref, cnt_ref, comp_ref, basesf_ref, out_ref):
    et = et_ref[0]
    iot = lax.broadcasted_iota(jnp.int32, (1, R), 1)
    onehot = (et == iot).astype(jnp.float32)
    cs = jnp.dot(onehot, comp_ref[...],
                 preferred_element_type=jnp.float32)
    xsb = jnp.dot(xs_ref[...], basesf_ref[...],
                  preferred_element_type=jnp.float32)
    acc = xsb[:, 0:D] * cs[:, 0:1]
    for b in range(1, NB):
        acc = acc + xsb[:, b * D:(b + 1) * D] * cs[:, b:b + 1]
    norm = 1.0 / jnp.maximum(cnt_ref[0], 1.0)
    out_ref[...] = acc * norm


def _tc_msg(xs, et3, cnt3, comp, basesf):
    return pl.pallas_call(
        _msg_body,
        grid=(E // _BE,),
        in_specs=[
            pl.BlockSpec((_BE, D), lambda i: (i, 0)),
            pl.BlockSpec((1, _BE, 1), lambda i: (i, 0, 0)),
            pl.BlockSpec((1, _BE, 1), lambda i: (i, 0, 0)),
            pl.BlockSpec((R, NB), lambda i: (0, 0)),
            pl.BlockSpec((D, NB * D), lambda i: (0, 0)),
        ],
        out_specs=pl.BlockSpec((_BE, D), lambda i: (i, 0)),
        out_shape=jax.ShapeDtypeStruct((E, D), jnp.float32),
    )(xs, et3, cnt3, comp, basesf)


_BN = 2048


def _h_body(agg_ref, x_ref, root_ref, bias_ref, out_ref):
    xr = jnp.dot(x_ref[...], root_ref[...],
                 preferred_element_type=jnp.float32)
    out_ref[...] = jnp.maximum(agg_ref[...] + xr + bias_ref[...], 0.0)


def _tc_h(agg, x, root, bias2):
    return pl.pallas_call(
        _h_body,
        grid=(N // _BN,),
        in_specs=[
            pl.BlockSpec((_BN, D), lambda i: (i, 0)),
            pl.BlockSpec((_BN, D), lambda i: (i, 0)),
            pl.BlockSpec((D, D), lambda i: (0, 0)),
            pl.BlockSpec((1, D), lambda i: (0, 0)),
        ],
        out_specs=pl.BlockSpec((_BN, D), lambda i: (i, 0)),
        out_shape=jax.ShapeDtypeStruct((N, D), jnp.float32),
    )(agg, x, root, bias2)


def _cos_body(p2_ref, tgt_ref, out_ref):
    pooled = p2_ref[0:B, :] + p2_ref[B:2 * B, :]
    tgt = tgt_ref[...]
    num = jnp.sum(pooled * tgt, axis=1, keepdims=True)
    den = jnp.sqrt(jnp.sum(pooled * pooled, axis=1, keepdims=True)) * \
        jnp.sqrt(jnp.sum(tgt * tgt, axis=1, keepdims=True))
    out_ref[...] = num / jnp.maximum(den, 1e-8)


def _tc_cos(pooled2, tgt):
    return pl.pallas_call(
        _cos_body,
        out_shape=jax.ShapeDtypeStruct((B, 1), jnp.float32),
    )(pooled2, tgt)


def kernel(anchor_ids, var_ids, edge_index, edge_type, batch_vec,
           target_nodes, node_emb, mode_emb, comp, bases, root, bias):
    anchor_ids = anchor_ids.astype(jnp.int32)
    target_nodes = target_nodes.astype(jnp.int32)
    src = edge_index[0].astype(jnp.int32)
    dst = edge_index[1].astype(jnp.int32)
    et = edge_type.astype(jnp.int32)
    bv = batch_vec.astype(jnp.int32)

    zeros_rows = jnp.zeros((CH, D), jnp.float32)
    zeros_flat = jnp.zeros((_ZPT,), jnp.float32)
    ones_v = jnp.ones((CH,), jnp.float32)

    gidx = jnp.concatenate([anchor_ids.reshape(-1), target_nodes])
    rows = _row_gather(node_emb, gidx.reshape(NT, 6, CH), B * 3)
    anch = rows[:B * N_ANCHORS].reshape(B, N_ANCHORS, D)
    tgt = rows[B * N_ANCHORS:]
    onehot3 = (var_ids[:, None] ==
               jnp.arange(mode_emb.shape[0], dtype=var_ids.dtype)[None, :])
    var3 = jnp.dot(onehot3.astype(jnp.float32), mode_emb)
    x = jnp.concatenate(
        [anch, jnp.broadcast_to(var3[None], (B, N_VARS, D))],
        axis=1).reshape(N, D)

    xs = _row_gather(x, src.reshape(NT, 6, CH), E)

    comb = dst * R + et
    cparts = []
    for g in range(_CPARTS):
        in_part = (comb >= g * _PC) & (comb < (g + 1) * _PC)
        cparts.append(jnp.where(in_part, comb - g * _PC, _PC))
    cnt_idx = jnp.stack(cparts).reshape(_CPARTS * NS, 12, CH)
    cnt_full = _pair_counts(cnt_idx, zeros_flat, ones_v)
    cnt_e = _elem_gather(cnt_full, comb.reshape(NT, 6, CH))

    basesf = jnp.transpose(bases, (1, 0, 2)).reshape(D, NB * D)
    msg = _tc_msg(xs, et.reshape(E // _BE, _BE, 1),
                  cnt_e.reshape(E // _BE, _BE, 1), comp, basesf)

    aparts = []
    for g in range(_APARTS):
        loc = dst - g * _PART
        aparts.append(jnp.where((dst >= g * _PART) & (dst < (g + 1) * _PART),
                                loc, _PART))
    agg_idx = jnp.stack(aparts).reshape(_APARTS * NS, 12, CH)
    agg = _agg_scatter(msg, agg_idx, zeros_rows).reshape(N, D)

    h = _tc_h(agg, x, root, bias.reshape(1, D))

    pparts = []
    for q in range(2):
        loc = bv - q * _BHALF
        pparts.append(jnp.where((bv >= q * _BHALF) & (bv < (q + 1) * _BHALF),
                                loc, _BHALF))
    pool_idx = jnp.stack(pparts).reshape(2 * NT, 10, CH)
    pooled2 = _pool_scatter(h, pool_idx, zeros_rows)

    out2 = _tc_cos(pooled2, tgt)
    return out2.reshape(B)


# speedup vs baseline: 55.7258x; 1.0184x over previous
import functools

import jax
import jax.numpy as jnp
from jax import lax
from jax.experimental import pallas as pl
from jax.experimental.pallas import tpu as pltpu
from jax.experimental.pallas import tpu_sc as plsc

B = 8192
N_ANCHORS = 2
N_VARS = 3
N = B * (N_ANCHORS + N_VARS)
E = 24576
D = 128
R = 64
NB = 10

NC = 2
NS = 16
NT = NC * NS
CH = 128

_MESH = plsc.VectorSubcoreMesh(core_axis_name="c", subcore_axis_name="s")


def _row_gather(table, idx_rows, n_out):
    nchunks = idx_rows.shape[1]
    per_tile = nchunks * CH

    @functools.partial(
        pl.kernel,
        out_type=jax.ShapeDtypeStruct((n_out, D), jnp.float32),
        mesh=_MESH,
        scratch_types=[
            pltpu.VMEM((nchunks, CH), jnp.int32),
            pltpu.VMEM((per_tile, D), jnp.float32),
        ],
    )
    def k(table_hbm, idx_hbm, out_hbm, idx_v, buf_v):
        wid = lax.axis_index("c") * NS + lax.axis_index("s")
        pltpu.sync_copy(idx_hbm.at[wid], idx_v)
        for j in range(nchunks):
            pltpu.sync_copy(table_hbm.at[idx_v.at[j]],
                            buf_v.at[pl.ds(j * CH, CH)])
        pltpu.sync_copy(buf_v, out_hbm.at[pl.ds(wid * per_tile, per_tile)])

    return k(table, idx_rows)


_CPARTS = 4
_PC = N * R // _CPARTS
_ZPT = (_PC + CH) // NS


def _pair_counts(cnt_idx, zeros_flat, ones_v):
    @functools.partial(
        pl.kernel,
        out_type=jax.ShapeDtypeStruct((N * R,), jnp.float32),
        mesh=_MESH,
        scratch_types=[
            pltpu.VMEM_SHARED((_PC + CH,), jnp.float32),
            pltpu.VMEM((E // NS // CH, CH), jnp.int32),
            pltpu.VMEM((CH,), jnp.float32),
            pltpu.VMEM((_ZPT,), jnp.float32),
        ],
    )
    def k(cidx_hbm, zflat_hbm, ones_hbm, cnt_hbm, shared, idx_v, ones_vv,
          buf_v):
        c = lax.axis_index("c")
        t = lax.axis_index("s")
        nchunks = E // NS // CH
        pltpu.sync_copy(ones_hbm, ones_vv)
        pltpu.sync_copy(zflat_hbm, buf_v)
        for p in range(_CPARTS // 2):
            g = c * (_CPARTS // 2) + p
            pltpu.sync_copy(buf_v, shared.at[pl.ds(t * _ZPT, _ZPT)])
            plsc.subcore_barrier()
            pltpu.sync_copy(cidx_hbm.at[g * NS + t], idx_v)
            for j in range(nchunks):
                pltpu.sync_copy(ones_vv, shared.at[idx_v.at[j]], add=True)
            plsc.subcore_barrier()
            per_tile = _PC // NS
            pltpu.sync_copy(shared.at[pl.ds(t * per_tile, per_tile)],
                            buf_v.at[pl.ds(0, per_tile)])
            pltpu.sync_copy(
                buf_v.at[pl.ds(0, per_tile)],
                cnt_hbm.at[pl.ds(g * _PC + t * per_tile, per_tile)])
            plsc.subcore_barrier()
            pltpu.sync_copy(zflat_hbm, buf_v)

    return k(cnt_idx, zeros_flat, ones_v)


def _elem_gather(table, idx_rows):
    nchunks = idx_rows.shape[1]
    per_tile = nchunks * CH

    @functools.partial(
        pl.kernel,
        out_type=jax.ShapeDtypeStruct((E,), jnp.float32),
        mesh=_MESH,
        scratch_types=[
            pltpu.VMEM((nchunks, CH), jnp.int32),
            pltpu.VMEM((per_tile,), jnp.float32),
        ],
    )
    def k(table_hbm, idx_hbm, out_hbm, idx_v, buf_v):
        wid = lax.axis_index("c") * NS + lax.axis_index("s")
        pltpu.sync_copy(idx_hbm.at[wid], idx_v)
        for j in range(nchunks):
            pltpu.sync_copy(table_hbm.at[idx_v.at[j]],
                            buf_v.at[pl.ds(j * CH, CH)])
        pltpu.sync_copy(buf_v, out_hbm.at[pl.ds(wid * per_tile, per_tile)])

    return k(table, idx_rows)


_APARTS = 8
_PART = N // _APARTS
_PROWS = _PART + CH
_ZROWS = _PROWS // NS
_RPT = _PART // NS


def _agg_scatter(msg, agg_idx, zeros_rows):
    @functools.partial(
        pl.kernel,
        out_type=jax.ShapeDtypeStruct((N // 64, 64, D), jnp.float32),
        mesh=_MESH,
        scratch_types=[
            pltpu.VMEM_SHARED((_PROWS, D), jnp.float32),
            pltpu.VMEM((E // NS // CH, CH), jnp.int32),
            pltpu.VMEM((CH, D), jnp.float32),
            pltpu.VMEM((CH, D), jnp.float32),
            pltpu.VMEM((CH, D), jnp.float32),
            pltpu.VMEM((CH, D), jnp.float32),
            pltpu.SemaphoreType.DMA,
            pltpu.SemaphoreType.DMA,
        ],
    )
    def k(msg_hbm, aidx_hbm, zrows_hbm, h_hbm,
          acc, idx_v, data_v, data2_v, zbuf_v, rbuf_v, sem0, sem1):
        c = lax.axis_index("c")
        t = lax.axis_index("s")
        nchunks = E // NS // CH
        pltpu.sync_copy(zrows_hbm, zbuf_v)
        for p in range(_APARTS // 2):
            g = c * (_APARTS // 2) + p
            for j in range(_ZROWS // CH):
                pltpu.sync_copy(zbuf_v,
                                acc.at[pl.ds(t * _ZROWS + j * CH, CH)])
            if _ZROWS % CH:
                pltpu.sync_copy(
                    zbuf_v.at[pl.ds(0, _ZROWS % CH)],
                    acc.at[pl.ds(t * _ZROWS + (_ZROWS // CH) * CH,
                                 _ZROWS % CH)])
            plsc.subcore_barrier()
            pltpu.sync_copy(aidx_hbm.at[g * NS + t], idx_v)
            bufs = (data_v, data2_v)
            sems = (sem0, sem1)
            cp = pltpu.async_copy(
                msg_hbm.at[pl.ds(t * (E // NS), CH)], bufs[0], sems[0])
            for j in range(nchunks):
                cp.wait()
                if j + 1 < nchunks:
                    cp = pltpu.async_copy(
                        msg_hbm.at[pl.ds(t * (E // NS) + (j + 1) * CH, CH)],
                        bufs[(j + 1) % 2], sems[(j + 1) % 2])
                pltpu.sync_copy(bufs[j % 2], acc.at[idx_v.at[j]], add=True)
            plsc.subcore_barrier()
            for j in range(_RPT // 64):
                pltpu.sync_copy(acc.at[pl.ds(t * _RPT + j * 64, 64)],
                                rbuf_v.at[pl.ds(0, 64)])
                pltpu.sync_copy(rbuf_v.at[pl.ds(0, 64)],
                                h_hbm.at[(g * NS + t) * (_RPT // 64) + j])
            plsc.subcore_barrier()

    return k(msg, agg_idx, zeros_rows)


_BHALF = B // 2
_BROWS = _BHALF + CH
_BZ = _BROWS // NS


def _pool_scatter(h, pool_idx, zeros_rows):
    @functools.partial(
        pl.kernel,
        out_type=jax.ShapeDtypeStruct((2 * B, D), jnp.float32),
        mesh=_MESH,
        scratch_types=[
            pltpu.VMEM_SHARED((_BROWS, D), jnp.float32),
            pltpu.VMEM((N // 2 // NS // CH, CH), jnp.int32),
            pltpu.VMEM((CH, D), jnp.float32),
            pltpu.VMEM((CH, D), jnp.float32),
            pltpu.VMEM((CH, D), jnp.float32),
            pltpu.SemaphoreType.DMA,
            pltpu.SemaphoreType.DMA,
        ],
    )
    def k(h_hbm, pidx_hbm, zrows_hbm, out_hbm, pool, idx_v, data_v, data2_v,
          zbuf_v, sem0, sem1):
        c = lax.axis_index("c")
        t = lax.axis_index("s")
        nchunks = N // 2 // NS // CH
        pltpu.sync_copy(zrows_hbm, zbuf_v)
        for q in range(2):
            for j in range(_BZ // CH):
                pltpu.sync_copy(zbuf_v,
                                pool.at[pl.ds(t * _BZ + j * CH, CH)])
            if _BZ % CH:
                pltpu.sync_copy(
                    zbuf_v.at[pl.ds(0, _BZ % CH)],
                    pool.at[pl.ds(t * _BZ + (_BZ // CH) * CH, _BZ % CH)])
            plsc.subcore_barrier()
            pltpu.sync_copy(pidx_hbm.at[q * NT + c * NS + t], idx_v)
            bufs = (data_v, data2_v)
            sems = (sem0, sem1)
            base = c * (N // 2) + t * nchunks * CH
            cp = pltpu.async_copy(h_hbm.at[pl.ds(base, CH)], bufs[0], sems[0])
            for j in range(nchunks):
                cp.wait()
                if j + 1 < nchunks:
                    cp = pltpu.async_copy(
                        h_hbm.at[pl.ds(base + (j + 1) * CH, CH)],
                        bufs[(j + 1) % 2], sems[(j + 1) % 2])
                pltpu.sync_copy(bufs[j % 2], pool.at[idx_v.at[j]], add=True)
            plsc.subcore_barrier()
            rpt = _BHALF // NS
            for j in range(rpt // CH):
                pltpu.sync_copy(pool.at[pl.ds(t * rpt + j * CH, CH)], data_v)
                pltpu.sync_copy(
                    data_v,
                    out_hbm.at[
                        pl.ds(c * B + q * _BHALF + t * rpt + j * CH, CH)])
            plsc.subcore_barrier()

    return k(h, pool_idx, zeros_rows)


_BE = 1024


def _msg_body(xs_ref, et_ref, cnt_ref, comp_ref, basesf_ref, out_ref):
    et = et_ref[0]
    iot = lax.broadcasted_iota(jnp.int32, (1, R), 1)
    onehot = (et == iot).astype(jnp.float32)
    cs = jnp.dot(onehot, comp_ref[...],
                 preferred_element_type=jnp.float32)
    xsb = jnp.dot(xs_ref[...], basesf_ref[...],
                  preferred_element_type=jnp.float32)
    acc = xsb[:, 0:D] * cs[:, 0:1]
    for b in range(1, NB):
        acc = acc + xsb[:, b * D:(b + 1) * D] * cs[:, b:b + 1]
    norm = 1.0 / jnp.maximum(cnt_ref[0], 1.0)
    out_ref[...] = acc * norm


def _tc_msg(xs, et3, cnt3, comp, basesf):
    return pl.pallas_call(
        _msg_body,
        grid=(E // _BE,),
        in_specs=[
            pl.BlockSpec((_BE, D), lambda i: (i, 0)),
            pl.BlockSpec((1, _BE, 1), lambda i: (i, 0, 0)),
            pl.BlockSpec((1, _BE, 1), lambda i: (i, 0, 0)),
            pl.BlockSpec((R, NB), lambda i: (0, 0)),
            pl.BlockSpec((D, NB * D), lambda i: (0, 0)),
        ],
        out_specs=pl.BlockSpec((_BE, D), lambda i: (i, 0)),
        out_shape=jax.ShapeDtypeStruct((E, D), jnp.float32),
    )(xs, et3, cnt3, comp, basesf)


_BN = 2048


def _h_body(agg_ref, x_ref, root_ref, bias_ref, out_ref):
    xr = jnp.dot(x_ref[...], root_ref[...],
                 preferred_element_type=jnp.float32)
    out_ref[...] = jnp.maximum(agg_ref[...] + xr + bias_ref[...], 0.0)


def _tc_h(agg, x, root, bias2):
    return pl.pallas_call(
        _h_body,
        grid=(N // _BN,),
        in_specs=[
            pl.BlockSpec((_BN, D), lambda i: (i, 0)),
            pl.BlockSpec((_BN, D), lambda i: (i, 0)),
            pl.BlockSpec((D, D), lambda i: (0, 0)),
            pl.BlockSpec((1, D), lambda i: (0, 0)),
        ],
        out_specs=pl.BlockSpec((_BN, D), lambda i: (i, 0)),
        out_shape=jax.ShapeDtypeStruct((N, D), jnp.float32),
    )(agg, x, root, bias2)


def _cos_body(p2_ref, tgt_ref, out_ref):
    pooled = p2_ref[0:B, :] + p2_ref[B:2 * B, :]
    tgt = tgt_ref[...]
    num = jnp.sum(pooled * tgt, axis=1, keepdims=True)
    den = jnp.sqrt(jnp.sum(pooled * pooled, axis=1, keepdims=True)) * \
        jnp.sqrt(jnp.sum(tgt * tgt, axis=1, keepdims=True))
    out_ref[...] = num / jnp.maximum(den, 1e-8)


def _tc_cos(pooled2, tgt):
    return pl.pallas_call(
        _cos_body,
        out_shape=jax.ShapeDtypeStruct((B, 1), jnp.float32),
    )(pooled2, tgt)


def kernel(anchor_ids, var_ids, edge_index, edge_type, batch_vec,
           target_nodes, node_emb, mode_emb, comp, bases, root, bias):
    anchor_ids = anchor_ids.astype(jnp.int32)
    target_nodes = target_nodes.astype(jnp.int32)
    src = edge_index[0].astype(jnp.int32)
    dst = edge_index[1].astype(jnp.int32)
    et = edge_type.astype(jnp.int32)
    bv = batch_vec.astype(jnp.int32)

    zeros_rows = jnp.zeros((CH, D), jnp.float32)
    zeros_flat = jnp.zeros((_ZPT,), jnp.float32)
    ones_v = jnp.ones((CH,), jnp.float32)

    gidx = jnp.concatenate([anchor_ids.reshape(-1), target_nodes])
    rows = _row_gather(node_emb, gidx.reshape(NT, 6, CH), B * 3)
    anch = rows[:B * N_ANCHORS].reshape(B, N_ANCHORS, D)
    tgt = rows[B * N_ANCHORS:]
    onehot3 = (var_ids[:, None] ==
               jnp.arange(mode_emb.shape[0], dtype=var_ids.dtype)[None, :])
    var3 = jnp.dot(onehot3.astype(jnp.float32), mode_emb)
    x = jnp.concatenate(
        [anch, jnp.broadcast_to(var3[None], (B, N_VARS, D))],
        axis=1).reshape(N, D)

    xs = _row_gather(x, src.reshape(NT, 6, CH), E)

    comb = dst * R + et
    cparts = []
    for g in range(_CPARTS):
        in_part = (comb >= g * _PC) & (comb < (g + 1) * _PC)
        cparts.append(jnp.where(in_part, comb - g * _PC, _PC))
    cnt_idx = jnp.stack(cparts).reshape(_CPARTS * NS, 12, CH)
    cnt_full = _pair_counts(cnt_idx, zeros_flat, ones_v)
    cnt_e = _elem_gather(cnt_full, comb.reshape(NT, 6, CH))

    basesf = jnp.transpose(bases, (1, 0, 2)).reshape(D, NB * D)
    msg = _tc_msg(xs, et.reshape(E // _BE, _BE, 1),
                  cnt_e.reshape(E // _BE, _BE, 1), comp, basesf)

    aparts = []
    for g in range(_APARTS):
        loc = dst - g * _PART
        aparts.append(jnp.where((dst >= g * _PART) & (dst < (g + 1) * _PART),
                                loc, _PART))
    agg_idx = jnp.stack(aparts).reshape(_APARTS * NS, 12, CH)
    agg = _agg_scatter(msg, agg_idx, zeros_rows).reshape(N, D)

    h = _tc_h(agg, x, root, bias.reshape(1, D))

    pparts = []
    for q in range(2):
        loc = bv - q * _BHALF
        pparts.append(jnp.where((bv >= q * _BHALF) & (bv < (q + 1) * _BHALF),
                                loc, _BHALF))
    pool_idx = jnp.stack(pparts).reshape(2 * NT, 10, CH)
    pooled2 = _pool_scatter(h, pool_idx, zeros_rows)

    out2 = _tc_cos(pooled2, tgt)
    return out2.reshape(B)
